# baseline (device time: 9425 ns/iter reference)
import jax
import jax.numpy as jnp
from jax import lax
from jax.experimental import pallas as pl
from jax.experimental.pallas import tpu as pltpu

T = 512
D = 256
CH = 32
N_CH = T // CH


def _body(x_ref, dest_ref, out_ref, sorted_buf, recv_buf, send_sems,
          recv_sems):
    mx = lax.axis_index("x")
    my = lax.axis_index("y")
    mz = lax.axis_index("z")
    peer = (mx, 1 - my, mz)

    recv_buf[...] = jnp.zeros((T, D), jnp.bfloat16)

    bar = pltpu.get_barrier_semaphore()
    pl.semaphore_signal(bar, inc=1, device_id=peer,
                        device_id_type=pl.DeviceIdType.MESH)
    pl.semaphore_wait(bar, 1)

    i16 = jnp.int16
    bf = jnp.bfloat16
    x = x_ref[...].astype(bf)
    d16 = dest_ref[...].astype(i16)
    my16 = my.astype(i16)
    sm = d16 != my16
    ns = jnp.sum(sm.astype(jnp.int32))
    ns16 = ns.astype(i16)

    p = lax.broadcasted_iota(i16, (T, T), 0)
    icol = lax.broadcasted_iota(i16, (T, T), 1)
    dif = icol - p

    tri = jnp.where(dif > 0, bf(1), bf(0))
    sm_bf = jnp.where(sm, bf(1), bf(0))
    send_rank = jnp.dot(sm_bf, tri, preferred_element_type=jnp.float32)
    sr16 = send_rank.astype(i16)
    irow = lax.broadcasted_iota(i16, (1, T), 1)
    rank = jnp.where(sm, sr16, ns16 + (irow - sr16))

    P = jnp.where(rank == p, bf(1), bf(0))
    sorted_buf[...] = jnp.dot(
        P, x, preferred_element_type=jnp.float32).astype(bf)

    def rdma(c):
        return pltpu.make_async_remote_copy(
            src_ref=sorted_buf.at[pl.ds(c * CH, CH)],
            dst_ref=recv_buf.at[pl.ds(c * CH, CH)],
            send_sem=send_sems.at[c],
            recv_sem=recv_sems.at[c],
            device_id=peer,
            device_id_type=pl.DeviceIdType.MESH,
        )

    for c in range(N_CH):
        @pl.when(c * CH < ns)
        def _(c=c):
            rdma(c).start()

    is0 = my == 0
    dA = jnp.where(is0, ns, 0).astype(i16)
    dB = jnp.where(is0, ns - T, 0).astype(i16)
    send_col = icol >= ns16
    A = jnp.where((dif == dA) & send_col, bf(1), bf(0))
    B = jnp.where((dif == dB) & (~send_col), bf(1), bf(0))
    ax = jnp.dot(A, sorted_buf[...], preferred_element_type=jnp.float32)

    for c in range(N_CH):
        @pl.when(c * CH < ns)
        def _(c=c):
            rdma(c).wait_recv()

    out_ref[...] = ax + jnp.dot(B, recv_buf[...],
                                preferred_element_type=jnp.float32)

    for c in range(N_CH):
        @pl.when(c * CH < ns)
        def _(c=c):
            rdma(c).wait_send()


def kernel(x, dest):
    return pl.pallas_call(
        _body,
        out_shape=jax.ShapeDtypeStruct((T, D), jnp.float32),
        in_specs=[
            pl.BlockSpec(memory_space=pltpu.VMEM),
            pl.BlockSpec(memory_space=pltpu.VMEM),
        ],
        out_specs=pl.BlockSpec(memory_space=pltpu.VMEM),
        scratch_shapes=[
            pltpu.VMEM((T, D), jnp.bfloat16),
            pltpu.VMEM((T, D), jnp.bfloat16),
            pltpu.SemaphoreType.DMA((N_CH,)),
            pltpu.SemaphoreType.DMA((N_CH,)),
        ],
        compiler_params=pltpu.CompilerParams(collective_id=0),
    )(x, dest.reshape(1, T))


# device time: 9414 ns/iter; 1.0012x vs baseline; 1.0012x over previous
import jax
import jax.numpy as jnp
from jax import lax
from jax.experimental import pallas as pl
from jax.experimental.pallas import tpu as pltpu

T = 512
D = 256
CH = 32
N_CH = T // CH


def _body(x_ref, dest_ref, out_ref, sorted_buf, recv_buf, send_sems,
          recv_sems):
    mx = lax.axis_index("x")
    my = lax.axis_index("y")
    mz = lax.axis_index("z")
    peer = (mx, 1 - my, mz)
    i16 = jnp.int16
    bf = jnp.bfloat16

    recv_buf[...] = jnp.zeros((T, D), bf)

    bar = pltpu.get_barrier_semaphore()
    pl.semaphore_signal(bar, inc=1, device_id=peer,
                        device_id_type=pl.DeviceIdType.MESH)

    x = x_ref[...].astype(bf)
    d16 = dest_ref[...].astype(i16)
    sm = d16 != my.astype(i16)
    ns = jnp.sum(sm.astype(jnp.int32))
    ns16 = ns.astype(i16)

    p = lax.broadcasted_iota(i16, (T, T), 0)
    icol = lax.broadcasted_iota(i16, (T, T), 1)
    dif = icol - p

    tri = jnp.where(dif > 0, bf(1), bf(0))
    sm_bf = jnp.where(sm, bf(1), bf(0))
    send_rank = jnp.dot(sm_bf, tri, preferred_element_type=jnp.float32)
    sr16 = send_rank.astype(i16)
    irow = lax.broadcasted_iota(i16, (1, T), 1)
    rank = jnp.where(sm, sr16, ns16 + (irow - sr16))

    P = jnp.where(rank == p, bf(1), bf(0))
    sorted_buf[...] = jnp.dot(
        P, x, preferred_element_type=jnp.float32).astype(bf)

    is0 = my == 0
    dA = jnp.where(is0, -ns, 0).astype(i16)
    A = jnp.where(((rank + dA) == p) & (~sm), bf(1), bf(0))
    ax = jnp.dot(A, x, preferred_element_type=jnp.float32)

    dB = jnp.where(is0, ns - T, 0).astype(i16)
    B = jnp.where((dif == dB) & (icol < ns16), bf(1), bf(0))

    pl.semaphore_wait(bar, 1)

    def rdma(c):
        return pltpu.make_async_remote_copy(
            src_ref=sorted_buf.at[pl.ds(c * CH, CH)],
            dst_ref=recv_buf.at[pl.ds(c * CH, CH)],
            send_sem=send_sems.at[c],
            recv_sem=recv_sems.at[c],
            device_id=peer,
            device_id_type=pl.DeviceIdType.MESH,
        )

    for c in range(N_CH):
        @pl.when(c * CH < ns)
        def _(c=c):
            rdma(c).start()

    for c in range(N_CH):
        @pl.when(c * CH < ns)
        def _(c=c):
            rdma(c).wait_recv()

    out_ref[...] = ax + jnp.dot(B, recv_buf[...],
                                preferred_element_type=jnp.float32)

    for c in range(N_CH):
        @pl.when(c * CH < ns)
        def _(c=c):
            rdma(c).wait_send()


def kernel(x, dest):
    return pl.pallas_call(
        _body,
        out_shape=jax.ShapeDtypeStruct((T, D), jnp.float32),
        in_specs=[
            pl.BlockSpec(memory_space=pltpu.VMEM),
            pl.BlockSpec(memory_space=pltpu.VMEM),
        ],
        out_specs=pl.BlockSpec(memory_space=pltpu.VMEM),
        scratch_shapes=[
            pltpu.VMEM((T, D), jnp.bfloat16),
            pltpu.VMEM((T, D), jnp.bfloat16),
            pltpu.SemaphoreType.DMA((N_CH,)),
            pltpu.SemaphoreType.DMA((N_CH,)),
        ],
        compiler_params=pltpu.CompilerParams(collective_id=0),
    )(x, dest.reshape(1, T))


# device time: 9239 ns/iter; 1.0201x vs baseline; 1.0189x over previous
import jax
import jax.numpy as jnp
from jax import lax
from jax.experimental import pallas as pl
from jax.experimental.pallas import tpu as pltpu

T = 512
D = 256
CH = 32
N_CH = T // CH


def _body(x_ref, dest_ref, out_ref, sorted_buf, recv_buf, send_sems,
          recv_sems):
    mx = lax.axis_index("x")
    my = lax.axis_index("y")
    mz = lax.axis_index("z")
    peer = (mx, 1 - my, mz)
    i16 = jnp.int16
    bf = jnp.bfloat16

    recv_buf[...] = jnp.zeros((T, D), bf)

    bar = pltpu.get_barrier_semaphore()
    pl.semaphore_signal(bar, inc=1, device_id=peer,
                        device_id_type=pl.DeviceIdType.MESH)

    x = x_ref[...].astype(bf)
    d16 = dest_ref[...].reshape(1, T).astype(i16)
    sm = d16 != my.astype(i16)
    ns = jnp.sum(sm.astype(jnp.int32))
    ns16 = ns.astype(i16)

    p = lax.broadcasted_iota(i16, (T, T), 0)
    icol = lax.broadcasted_iota(i16, (T, T), 1)
    dif = icol - p

    tri = jnp.where(dif > 0, bf(1), bf(0))
    sm_bf = jnp.where(sm, bf(1), bf(0))
    send_rank = jnp.dot(sm_bf, tri, preferred_element_type=jnp.float32)
    sr16 = send_rank.astype(i16)
    irow = lax.broadcasted_iota(i16, (1, T), 1)
    rank = jnp.where(sm, sr16, ns16 + (irow - sr16))

    P = jnp.where(rank == p, bf(1), bf(0))

    is0 = my == 0
    dA = jnp.where(is0, -ns, 0).astype(i16)
    A = jnp.where(((rank + dA) == p) & (~sm), bf(1), bf(0))
    ax = jnp.dot(A, x, preferred_element_type=jnp.float32)

    dB = jnp.where(is0, ns - T, 0).astype(i16)
    B = jnp.where((dif == dB) & (icol < ns16), bf(1), bf(0))

    pl.semaphore_wait(bar, 1)

    def rdma(c):
        return pltpu.make_async_remote_copy(
            src_ref=sorted_buf.at[pl.ds(c * CH, CH)],
            dst_ref=recv_buf.at[pl.ds(c * CH, CH)],
            send_sem=send_sems.at[c],
            recv_sem=recv_sems.at[c],
            device_id=peer,
            device_id_type=pl.DeviceIdType.MESH,
        )

    G = 4
    GR = T // G
    GC = GR // CH
    for g in range(G):
        sorted_buf[pl.ds(g * GR, GR), :] = jnp.dot(
            P[g * GR:(g + 1) * GR, :], x,
            preferred_element_type=jnp.float32).astype(bf)
        for cc in range(GC):
            c = g * GC + cc
            @pl.when(c * CH < ns)
            def _(c=c):
                rdma(c).start()

    for c in range(N_CH):
        @pl.when(c * CH < ns)
        def _(c=c):
            rdma(c).wait_recv()

    out_ref[...] = ax + jnp.dot(B, recv_buf[...],
                                preferred_element_type=jnp.float32)

    for c in range(N_CH):
        @pl.when(c * CH < ns)
        def _(c=c):
            rdma(c).wait_send()


def kernel(x, dest):
    return pl.pallas_call(
        _body,
        out_shape=jax.ShapeDtypeStruct((T, D), jnp.float32),
        in_specs=[
            pl.BlockSpec(memory_space=pltpu.VMEM),
            pl.BlockSpec(memory_space=pltpu.VMEM),
        ],
        out_specs=pl.BlockSpec(memory_space=pltpu.VMEM),
        scratch_shapes=[
            pltpu.VMEM((T, D), jnp.bfloat16),
            pltpu.VMEM((T, D), jnp.bfloat16),
            pltpu.SemaphoreType.DMA((N_CH,)),
            pltpu.SemaphoreType.DMA((N_CH,)),
        ],
        compiler_params=pltpu.CompilerParams(collective_id=0),
    )(x, dest)
